# baseline (speedup 1.0000x reference)
import functools

import jax
import jax.numpy as jnp
from jax import lax
from jax.experimental import pallas as pl
from jax.experimental.pallas import tpu as pltpu
from jax.experimental.pallas import tpu_sc as plsc

NC = 2
NS = 16
NW = NC * NS
CH = 80
BK = 2000


def _silu(x):
    return x * jax.nn.sigmoid(x)


def _tables_body(h_ref, wa_ref, wb_ref, a_ref, b_ref):
    h = h_ref[...]
    a_ref[...] = jnp.dot(h, wa_ref[...],
                         preferred_element_type=jnp.float32).astype(jnp.bfloat16)
    b_ref[...] = jnp.dot(h, wb_ref[...],
                         preferred_element_type=jnp.float32).astype(jnp.bfloat16)


def _tables(h, w1a, w1b):
    n, d = h.shape
    return pl.pallas_call(
        _tables_body,
        out_shape=[jax.ShapeDtypeStruct((n, d), jnp.bfloat16),
                   jax.ShapeDtypeStruct((n, d), jnp.bfloat16)],
    )(h, w1a, w1b)


def _chunk(epw):
    for ch in (80, 40, 16, 8):
        if epw % ch == 0:
            return ch
    raise ValueError(epw)


def _gather(ha, hb, row, col):
    n, d = ha.shape
    e = row.shape[0]
    epw = e // NW
    CH = _chunk(epw)
    nchunk = epw // CH
    mesh = plsc.VectorSubcoreMesh(core_axis_name="c", subcore_axis_name="s")

    def body(ha_hbm, hb_hbm, row_hbm, col_hbm, outa, outb,
             idxa, idxb, bufa, bufb, sems, semw):
        c = lax.axis_index("c")
        s = lax.axis_index("s")
        base = (c * NS + s) * epw
        pltpu.sync_copy(row_hbm.at[c, s], idxa)
        pltpu.sync_copy(col_hbm.at[c, s], idxb)

        def fetch(i, slot):
            pltpu.async_copy(ha_hbm.at[idxa.at[i]], bufa.at[slot],
                             sems.at[slot])
            pltpu.async_copy(hb_hbm.at[idxb.at[i]], bufb.at[slot],
                             sems.at[slot])

        def drain(i, slot):
            pltpu.make_async_copy(ha_hbm.at[idxa.at[i]], bufa.at[slot],
                                  sems.at[slot]).wait()
            pltpu.make_async_copy(hb_hbm.at[idxb.at[i]], bufb.at[slot],
                                  sems.at[slot]).wait()
            off = base + i * CH
            pltpu.async_copy(bufa.at[slot], outa.at[pl.ds(off, CH)],
                             semw.at[slot])
            pltpu.async_copy(bufb.at[slot], outb.at[pl.ds(off, CH)],
                             semw.at[slot])

        def wait_write(i, slot):
            off = base + i * CH
            pltpu.make_async_copy(bufa.at[slot], outa.at[pl.ds(off, CH)],
                                  semw.at[slot]).wait()
            pltpu.make_async_copy(bufb.at[slot], outb.at[pl.ds(off, CH)],
                                  semw.at[slot]).wait()

        fetch(0, 0)

        def step(i, carry):
            slot = lax.rem(i, 2)
            nslot = 1 - slot

            @pl.when(i + 1 < nchunk)
            def _():
                @pl.when(i >= 1)
                def _():
                    wait_write(i - 1, nslot)

                fetch(i + 1, nslot)

            drain(i, slot)
            return carry

        lax.fori_loop(0, nchunk, step, 0)
        for k in (nchunk - 2, nchunk - 1):
            wait_write(k, k % 2)

    fn = pl.kernel(
        body,
        out_type=[jax.ShapeDtypeStruct((e, d), jnp.bfloat16),
                  jax.ShapeDtypeStruct((e, d), jnp.bfloat16)],
        mesh=mesh,
        compiler_params=pltpu.CompilerParams(use_tc_tiling_on_sc=False),
        scratch_types=[
            pltpu.VMEM((nchunk, CH), jnp.int32),
            pltpu.VMEM((nchunk, CH), jnp.int32),
            pltpu.VMEM((2, CH, d), jnp.bfloat16),
            pltpu.VMEM((2, CH, d), jnp.bfloat16),
            pltpu.SemaphoreType.DMA((2,)),
            pltpu.SemaphoreType.DMA((2,)),
        ],
    )
    row4 = row.reshape(NC, NS, nchunk, CH)
    col4 = col.reshape(NC, NS, nchunk, CH)
    return fn(ha, hb, row4, col4)


def _mlp_body(pa_ref, pb_ref, cd_ref, w1c_ref, b1_ref, w2_ref, b2_ref,
              wc1_ref, bc1_ref, wc2_ref, out_ref):
    cd = cd_ref[...]
    radial = jnp.sum(cd * cd, axis=1, keepdims=True)
    pre = (pa_ref[...].astype(jnp.float32)
           + pb_ref[...].astype(jnp.float32))
    x = pre + radial * w1c_ref[...] + b1_ref[...]
    x = _silu(x)
    b16 = lambda v: v.astype(jnp.bfloat16)
    x = _silu(jnp.dot(b16(x), b16(w2_ref[...]),
                      preferred_element_type=jnp.float32) + b2_ref[...])
    x = _silu(jnp.dot(b16(x), b16(wc1_ref[...]),
                      preferred_element_type=jnp.float32) + bc1_ref[...])
    s = jnp.dot(x, wc2_ref[...], preferred_element_type=jnp.float32)
    t = jnp.clip(cd * s, -100.0, 100.0)
    ones = jnp.ones((t.shape[0], 1), jnp.float32)
    zeros = jnp.zeros((t.shape[0], 4), jnp.float32)
    out_ref[...] = jnp.concatenate([t, ones, zeros], axis=1)


def _mlp(pa, pb, cd, w1c, b1, w2, b2, wc1, bc1, wc2):
    e, d = pa.shape
    grid = (e // BK,)
    edge = lambda i: (i, 0)
    whole = lambda i: (0, 0)
    return pl.pallas_call(
        _mlp_body,
        grid=grid,
        in_specs=[
            pl.BlockSpec((BK, d), edge),
            pl.BlockSpec((BK, d), edge),
            pl.BlockSpec((BK, 3), edge),
            pl.BlockSpec((1, d), whole),
            pl.BlockSpec((1, d), whole),
            pl.BlockSpec((d, d), whole),
            pl.BlockSpec((1, d), whole),
            pl.BlockSpec((d, d), whole),
            pl.BlockSpec((1, d), whole),
            pl.BlockSpec((d, 1), whole),
        ],
        out_specs=pl.BlockSpec((BK, 8), edge),
        out_shape=jax.ShapeDtypeStruct((e, 8), jnp.float32),
    )(pa, pb, cd, w1c, b1, w2, b2, wc1, bc1, wc2)


def _scatter(t8, row, zer):
    e = row.shape[0]
    n = zer.shape[0]
    eh = e // NC
    ept = eh // NS
    CH = _chunk(ept)
    nchunk = ept // CH
    mesh = plsc.VectorSubcoreMesh(core_axis_name="c", subcore_axis_name="s")

    def body(t8_hbm, row_hbm, zer_hbm, out_hbm, idxv, buf, acc, seml, sems):
        c = lax.axis_index("c")
        s = lax.axis_index("s")
        pltpu.sync_copy(row_hbm.at[c, s], idxv)

        @pl.when(s == 0)
        def _():
            pltpu.sync_copy(zer_hbm, acc)

        plsc.subcore_barrier()
        base = c * eh + s * ept

        def fetch(i, slot):
            off = base + i * CH
            pltpu.async_copy(t8_hbm.at[pl.ds(off, CH)], buf.at[slot],
                             seml.at[slot])

        def wait_fetch(i, slot):
            off = base + i * CH
            pltpu.make_async_copy(t8_hbm.at[pl.ds(off, CH)], buf.at[slot],
                                  seml.at[slot]).wait()

        def scat(i, slot):
            pltpu.async_copy(buf.at[slot], acc.at[idxv.at[i]],
                             sems.at[slot], add=True)

        def wait_scat(i, slot):
            pltpu.make_async_copy(buf.at[slot], acc.at[idxv.at[i]],
                                  sems.at[slot]).wait()

        fetch(0, 0)

        def step(i, carry):
            slot = lax.rem(i, 2)
            nslot = 1 - slot

            @pl.when(i + 1 < nchunk)
            def _():
                @pl.when(i >= 1)
                def _():
                    wait_scat(i - 1, nslot)

                fetch(i + 1, nslot)

            wait_fetch(i, slot)
            scat(i, slot)
            return carry

        lax.fori_loop(0, nchunk, step, 0)
        for k in (nchunk - 2, nchunk - 1):
            wait_scat(k, k % 2)
        plsc.subcore_barrier()

        @pl.when(s == 0)
        def _():
            pltpu.sync_copy(acc, out_hbm.at[c])

    fn = pl.kernel(
        body,
        out_type=jax.ShapeDtypeStruct((NC, n, 8), jnp.float32),
        mesh=mesh,
        compiler_params=pltpu.CompilerParams(use_tc_tiling_on_sc=False),
        scratch_types=[
            pltpu.VMEM((nchunk, CH), jnp.int32),
            pltpu.VMEM((2, CH, 8), jnp.float32),
            pltpu.VMEM_SHARED((n, 8), jnp.float32),
            pltpu.SemaphoreType.DMA((2,)),
            pltpu.SemaphoreType.DMA((2,)),
        ],
    )
    row4 = row.reshape(NC, NS, nchunk, CH)
    return fn(t8, row4, zer)


def _combine_body(p_ref, o_ref):
    p = p_ref[...]
    acc = jnp.sum(p, axis=0)
    cnt = jnp.maximum(acc[:, 3:4], 1.0)
    o_ref[...] = acc[:, 0:3] / cnt


def _combine(parts):
    _, n, _ = parts.shape
    return pl.pallas_call(
        _combine_body,
        out_shape=jax.ShapeDtypeStruct((n, 3), jnp.float32),
    )(parts)


NSLICE = 2


def kernel(h, coord_diff, edge_index, W1, b1, W2, b2, Wc1, bc1, Wc2):
    n, d = h.shape
    e = coord_diff.shape[0]
    row = edge_index[0]
    col = edge_index[1]
    w1a = W1[:d]
    w1b = W1[d:2 * d]
    w1c = W1[2 * d].reshape(1, d)
    ha, hb = _tables(h, w1a, w1b)
    zer = jnp.zeros((n, 8), jnp.float32)
    es = e // NSLICE
    parts = []
    for k in range(NSLICE):
        sl = slice(k * es, (k + 1) * es)
        pa, pb = _gather(ha, hb, row[sl], col[sl])
        t8 = _mlp(pa, pb, coord_diff[sl], w1c, b1.reshape(1, d), W2,
                  b2.reshape(1, d), Wc1, bc1.reshape(1, d), Wc2)
        parts.append(_scatter(t8, row[sl], zer))
    return _combine(jnp.concatenate(parts, axis=0))


# speedup vs baseline: 1.6930x; 1.6930x over previous
import functools

import jax
import jax.numpy as jnp
from jax import lax
from jax.experimental import pallas as pl
from jax.experimental.pallas import tpu as pltpu
from jax.experimental.pallas import tpu_sc as plsc

NC = 2
NS = 16
NW = NC * NS
CH = 80
BK = 2000


def _silu(x):
    return x * jax.nn.sigmoid(x)


def _tables_body(h_ref, wa_ref, wb_ref, a_ref, b_ref):
    h = h_ref[...]
    a_ref[...] = jnp.dot(h, wa_ref[...], preferred_element_type=jnp.float32)
    b_ref[...] = jnp.dot(h, wb_ref[...], preferred_element_type=jnp.float32)


def _tables(h, w1a, w1b):
    n, d = h.shape
    return pl.pallas_call(
        _tables_body,
        out_shape=[jax.ShapeDtypeStruct((n, d), jnp.float32),
                   jax.ShapeDtypeStruct((n, d), jnp.float32)],
    )(h, w1a, w1b)


def _chunk(epw):
    for ch in (80, 40, 16, 8):
        if epw % ch == 0:
            return ch
    raise ValueError(epw)


def _gather(ha, hb, row, col):
    n, d = ha.shape
    e = row.shape[0]
    epw = e // NW
    CH = _chunk(epw)
    nchunk = epw // CH
    mesh = plsc.VectorSubcoreMesh(core_axis_name="c", subcore_axis_name="s")

    def body(ha_hbm, hb_hbm, row_hbm, col_hbm, outa, outb,
             idxa, idxb, bufa, bufb, sems, semw):
        c = lax.axis_index("c")
        s = lax.axis_index("s")
        base = (c * NS + s) * epw
        pltpu.sync_copy(row_hbm.at[c, s], idxa)
        pltpu.sync_copy(col_hbm.at[c, s], idxb)

        def fetch(i, slot):
            pltpu.async_copy(ha_hbm.at[idxa.at[i]], bufa.at[slot],
                             sems.at[slot])
            pltpu.async_copy(hb_hbm.at[idxb.at[i]], bufb.at[slot],
                             sems.at[slot])

        def drain(i, slot):
            pltpu.make_async_copy(ha_hbm.at[idxa.at[i]], bufa.at[slot],
                                  sems.at[slot]).wait()
            pltpu.make_async_copy(hb_hbm.at[idxb.at[i]], bufb.at[slot],
                                  sems.at[slot]).wait()
            off = base + i * CH
            pltpu.async_copy(bufa.at[slot], outa.at[pl.ds(off, CH)],
                             semw.at[slot])
            pltpu.async_copy(bufb.at[slot], outb.at[pl.ds(off, CH)],
                             semw.at[slot])

        def wait_write(i, slot):
            off = base + i * CH
            pltpu.make_async_copy(bufa.at[slot], outa.at[pl.ds(off, CH)],
                                  semw.at[slot]).wait()
            pltpu.make_async_copy(bufb.at[slot], outb.at[pl.ds(off, CH)],
                                  semw.at[slot]).wait()

        fetch(0, 0)

        def step(i, carry):
            slot = lax.rem(i, 2)
            nslot = 1 - slot

            @pl.when(i + 1 < nchunk)
            def _():
                @pl.when(i >= 1)
                def _():
                    wait_write(i - 1, nslot)

                fetch(i + 1, nslot)

            drain(i, slot)
            return carry

        lax.fori_loop(0, nchunk, step, 0)
        for k in (nchunk - 2, nchunk - 1):
            wait_write(k, k % 2)

    fn = pl.kernel(
        body,
        out_type=[jax.ShapeDtypeStruct((e, d), jnp.float32),
                  jax.ShapeDtypeStruct((e, d), jnp.float32)],
        mesh=mesh,
        scratch_types=[
            pltpu.VMEM((nchunk, CH), jnp.int32),
            pltpu.VMEM((nchunk, CH), jnp.int32),
            pltpu.VMEM((2, CH, d), jnp.float32),
            pltpu.VMEM((2, CH, d), jnp.float32),
            pltpu.SemaphoreType.DMA((2,)),
            pltpu.SemaphoreType.DMA((2,)),
        ],
    )
    row4 = row.reshape(NC, NS, nchunk, CH)
    col4 = col.reshape(NC, NS, nchunk, CH)
    return fn(ha, hb, row4, col4)


def _mlp_body(pa_ref, pb_ref, cd_ref, w1c_ref, b1_ref, w2_ref, b2_ref,
              wc1_ref, bc1_ref, wc2_ref, out_ref):
    cd = cd_ref[...]
    radial = jnp.sum(cd * cd, axis=1, keepdims=True)
    x = pa_ref[...] + pb_ref[...] + radial * w1c_ref[...] + b1_ref[...]
    x = _silu(x)
    x = _silu(jnp.dot(x, w2_ref[...], preferred_element_type=jnp.float32)
              + b2_ref[...])
    x = _silu(jnp.dot(x, wc1_ref[...], preferred_element_type=jnp.float32)
              + bc1_ref[...])
    s = jnp.dot(x, wc2_ref[...], preferred_element_type=jnp.float32)
    t = jnp.clip(cd * s, -100.0, 100.0)
    ones = jnp.ones((t.shape[0], 1), jnp.float32)
    zeros = jnp.zeros((t.shape[0], 4), jnp.float32)
    out_ref[...] = jnp.concatenate([t, ones, zeros], axis=1)


def _mlp(pa, pb, cd, w1c, b1, w2, b2, wc1, bc1, wc2):
    e, d = pa.shape
    grid = (e // BK,)
    edge = lambda i: (i, 0)
    whole = lambda i: (0, 0)
    return pl.pallas_call(
        _mlp_body,
        grid=grid,
        in_specs=[
            pl.BlockSpec((BK, d), edge),
            pl.BlockSpec((BK, d), edge),
            pl.BlockSpec((BK, 3), edge),
            pl.BlockSpec((1, d), whole),
            pl.BlockSpec((1, d), whole),
            pl.BlockSpec((d, d), whole),
            pl.BlockSpec((1, d), whole),
            pl.BlockSpec((d, d), whole),
            pl.BlockSpec((1, d), whole),
            pl.BlockSpec((d, 1), whole),
        ],
        out_specs=pl.BlockSpec((BK, 8), edge),
        out_shape=jax.ShapeDtypeStruct((e, 8), jnp.float32),
    )(pa, pb, cd, w1c, b1, w2, b2, wc1, bc1, wc2)


def _scatter(t8, row, zer):
    e = row.shape[0]
    n = zer.shape[0]
    eh = e // NC
    ept = eh // NS
    CH = _chunk(ept)
    nchunk = ept // CH
    mesh = plsc.VectorSubcoreMesh(core_axis_name="c", subcore_axis_name="s")

    def body(t8_hbm, row_hbm, zer_hbm, out_hbm, idxv, buf, acc, seml, sems):
        c = lax.axis_index("c")
        s = lax.axis_index("s")
        pltpu.sync_copy(row_hbm.at[c, s], idxv)

        @pl.when(s == 0)
        def _():
            pltpu.sync_copy(zer_hbm, acc)

        plsc.subcore_barrier()
        base = c * eh + s * ept

        def fetch(i, slot):
            off = base + i * CH
            pltpu.async_copy(t8_hbm.at[pl.ds(off, CH)], buf.at[slot],
                             seml.at[slot])

        def wait_fetch(i, slot):
            off = base + i * CH
            pltpu.make_async_copy(t8_hbm.at[pl.ds(off, CH)], buf.at[slot],
                                  seml.at[slot]).wait()

        def scat(i, slot):
            pltpu.async_copy(buf.at[slot], acc.at[idxv.at[i]],
                             sems.at[slot], add=True)

        def wait_scat(i, slot):
            pltpu.make_async_copy(buf.at[slot], acc.at[idxv.at[i]],
                                  sems.at[slot]).wait()

        fetch(0, 0)

        def step(i, carry):
            slot = lax.rem(i, 2)
            nslot = 1 - slot

            @pl.when(i + 1 < nchunk)
            def _():
                @pl.when(i >= 1)
                def _():
                    wait_scat(i - 1, nslot)

                fetch(i + 1, nslot)

            wait_fetch(i, slot)
            scat(i, slot)
            return carry

        lax.fori_loop(0, nchunk, step, 0)
        for k in (nchunk - 2, nchunk - 1):
            wait_scat(k, k % 2)
        plsc.subcore_barrier()

        @pl.when(s == 0)
        def _():
            pltpu.sync_copy(acc, out_hbm.at[c])

    fn = pl.kernel(
        body,
        out_type=jax.ShapeDtypeStruct((NC, n, 8), jnp.float32),
        mesh=mesh,
        compiler_params=pltpu.CompilerParams(use_tc_tiling_on_sc=False),
        scratch_types=[
            pltpu.VMEM((nchunk, CH), jnp.int32),
            pltpu.VMEM((2, CH, 8), jnp.float32),
            pltpu.VMEM_SHARED((n, 8), jnp.float32),
            pltpu.SemaphoreType.DMA((2,)),
            pltpu.SemaphoreType.DMA((2,)),
        ],
    )
    row4 = row.reshape(NC, NS, nchunk, CH)
    return fn(t8, row4, zer)


def _combine_body(p_ref, o_ref):
    p = p_ref[...]
    acc = jnp.sum(p, axis=0)
    cnt = jnp.maximum(acc[:, 3:4], 1.0)
    o_ref[...] = acc[:, 0:3] / cnt


def _combine(parts):
    _, n, _ = parts.shape
    return pl.pallas_call(
        _combine_body,
        out_shape=jax.ShapeDtypeStruct((n, 3), jnp.float32),
    )(parts)


NSLICE = 5


def kernel(h, coord_diff, edge_index, W1, b1, W2, b2, Wc1, bc1, Wc2):
    n, d = h.shape
    e = coord_diff.shape[0]
    row = edge_index[0]
    col = edge_index[1]
    w1a = W1[:d]
    w1b = W1[d:2 * d]
    w1c = W1[2 * d].reshape(1, d)
    ha, hb = _tables(h, w1a, w1b)
    zer = jnp.zeros((n, 8), jnp.float32)
    es = e // NSLICE
    parts = []
    for k in range(NSLICE):
        sl = slice(k * es, (k + 1) * es)
        pa, pb = _gather(ha, hb, row[sl], col[sl])
        t8 = _mlp(pa, pb, coord_diff[sl], w1c, b1.reshape(1, d), W2,
                  b2.reshape(1, d), Wc1, bc1.reshape(1, d), Wc2)
        parts.append(_scatter(t8, row[sl], zer))
    return _combine(jnp.concatenate(parts, axis=0))


# speedup vs baseline: 1.7381x; 1.0266x over previous
import functools

import jax
import jax.numpy as jnp
from jax import lax
from jax.experimental import pallas as pl
from jax.experimental.pallas import tpu as pltpu
from jax.experimental.pallas import tpu_sc as plsc

NC = 2
NS = 16
NW = NC * NS
CH = 80
BK = 2000


def _silu(x):
    return x * jax.nn.sigmoid(x)


def _tables_body(h_ref, wa_ref, wb_ref, a_ref, b_ref):
    h = h_ref[...]
    a_ref[...] = jnp.dot(h, wa_ref[...], preferred_element_type=jnp.float32)
    b_ref[...] = jnp.dot(h, wb_ref[...], preferred_element_type=jnp.float32)


def _tables(h, w1a, w1b):
    n, d = h.shape
    return pl.pallas_call(
        _tables_body,
        out_shape=[jax.ShapeDtypeStruct((n, d), jnp.float32),
                   jax.ShapeDtypeStruct((n, d), jnp.float32)],
    )(h, w1a, w1b)


def _chunk(epw):
    for ch in (80, 40, 16, 8):
        if epw % ch == 0:
            return ch
    raise ValueError(epw)


def _gather(ha, hb, row, col):
    n, d = ha.shape
    e = row.shape[0]
    epw = e // NW
    CH = _chunk(epw)
    nchunk = epw // CH
    mesh = plsc.VectorSubcoreMesh(core_axis_name="c", subcore_axis_name="s")

    def body(ha_hbm, hb_hbm, row_hbm, col_hbm, outa, outb,
             idxa, idxb, bufa, bufb, sems, semw):
        c = lax.axis_index("c")
        s = lax.axis_index("s")
        base = (c * NS + s) * epw
        pltpu.sync_copy(row_hbm.at[c, s], idxa)
        pltpu.sync_copy(col_hbm.at[c, s], idxb)

        def fetch(i, slot):
            pltpu.async_copy(ha_hbm.at[idxa.at[i]], bufa.at[slot],
                             sems.at[slot])
            pltpu.async_copy(hb_hbm.at[idxb.at[i]], bufb.at[slot],
                             sems.at[slot])

        def drain(i, slot):
            pltpu.make_async_copy(ha_hbm.at[idxa.at[i]], bufa.at[slot],
                                  sems.at[slot]).wait()
            pltpu.make_async_copy(hb_hbm.at[idxb.at[i]], bufb.at[slot],
                                  sems.at[slot]).wait()
            off = base + i * CH
            pltpu.async_copy(bufa.at[slot], outa.at[pl.ds(off, CH)],
                             semw.at[slot])
            pltpu.async_copy(bufb.at[slot], outb.at[pl.ds(off, CH)],
                             semw.at[slot])

        def wait_write(i, slot):
            off = base + i * CH
            pltpu.make_async_copy(bufa.at[slot], outa.at[pl.ds(off, CH)],
                                  semw.at[slot]).wait()
            pltpu.make_async_copy(bufb.at[slot], outb.at[pl.ds(off, CH)],
                                  semw.at[slot]).wait()

        fetch(0, 0)

        def step(i, carry):
            slot = lax.rem(i, 2)
            nslot = 1 - slot

            @pl.when(i + 1 < nchunk)
            def _():
                @pl.when(i >= 1)
                def _():
                    wait_write(i - 1, nslot)

                fetch(i + 1, nslot)

            drain(i, slot)
            return carry

        lax.fori_loop(0, nchunk, step, 0)
        for k in (nchunk - 2, nchunk - 1):
            wait_write(k, k % 2)

    fn = pl.kernel(
        body,
        out_type=[jax.ShapeDtypeStruct((e, d), jnp.float32),
                  jax.ShapeDtypeStruct((e, d), jnp.float32)],
        mesh=mesh,
        scratch_types=[
            pltpu.VMEM((nchunk, CH), jnp.int32),
            pltpu.VMEM((nchunk, CH), jnp.int32),
            pltpu.VMEM((2, CH, d), jnp.float32),
            pltpu.VMEM((2, CH, d), jnp.float32),
            pltpu.SemaphoreType.DMA((2,)),
            pltpu.SemaphoreType.DMA((2,)),
        ],
    )
    row4 = row.reshape(NC, NS, nchunk, CH)
    col4 = col.reshape(NC, NS, nchunk, CH)
    return fn(ha, hb, row4, col4)


def _mlp_body(pa_ref, pb_ref, cd_ref, w1c_ref, b1_ref, w2_ref, b2_ref,
              wc1_ref, bc1_ref, wc2_ref, out_ref):
    cd = cd_ref[...]
    radial = jnp.sum(cd * cd, axis=1, keepdims=True)
    x = pa_ref[...] + pb_ref[...] + radial * w1c_ref[...] + b1_ref[...]
    x = _silu(x)
    x = _silu(jnp.dot(x, w2_ref[...], preferred_element_type=jnp.float32)
              + b2_ref[...])
    x = _silu(jnp.dot(x, wc1_ref[...], preferred_element_type=jnp.float32)
              + bc1_ref[...])
    s = jnp.dot(x, wc2_ref[...], preferred_element_type=jnp.float32)
    t = jnp.clip(cd * s, -100.0, 100.0)
    ones = jnp.ones((t.shape[0], 1), jnp.float32)
    zeros = jnp.zeros((t.shape[0], 4), jnp.float32)
    out_ref[...] = jnp.concatenate([t, ones, zeros], axis=1)


def _mlp(pa, pb, cd, w1c, b1, w2, b2, wc1, bc1, wc2):
    e, d = pa.shape
    grid = (e // BK,)
    edge = lambda i: (i, 0)
    whole = lambda i: (0, 0)
    return pl.pallas_call(
        _mlp_body,
        grid=grid,
        in_specs=[
            pl.BlockSpec((BK, d), edge),
            pl.BlockSpec((BK, d), edge),
            pl.BlockSpec((BK, 3), edge),
            pl.BlockSpec((1, d), whole),
            pl.BlockSpec((1, d), whole),
            pl.BlockSpec((d, d), whole),
            pl.BlockSpec((1, d), whole),
            pl.BlockSpec((d, d), whole),
            pl.BlockSpec((1, d), whole),
            pl.BlockSpec((d, 1), whole),
        ],
        out_specs=pl.BlockSpec((BK, 8), edge),
        out_shape=jax.ShapeDtypeStruct((e, 8), jnp.float32),
    )(pa, pb, cd, w1c, b1, w2, b2, wc1, bc1, wc2)


def _scatter(t8, row, zer):
    e = row.shape[0]
    n = zer.shape[0]
    eh = e // NC
    ept = eh // NS
    CH = _chunk(ept)
    nchunk = ept // CH
    mesh = plsc.VectorSubcoreMesh(core_axis_name="c", subcore_axis_name="s")

    def body(t8_hbm, row_hbm, zer_hbm, out_hbm, idxv, buf, acc, seml, sems):
        c = lax.axis_index("c")
        s = lax.axis_index("s")
        pltpu.sync_copy(row_hbm.at[c, s], idxv)

        @pl.when(s == 0)
        def _():
            pltpu.sync_copy(zer_hbm, acc)

        plsc.subcore_barrier()
        base = c * eh + s * ept

        def fetch(i, slot):
            off = base + i * CH
            pltpu.async_copy(t8_hbm.at[pl.ds(off, CH)], buf.at[slot],
                             seml.at[slot])

        def wait_fetch(i, slot):
            off = base + i * CH
            pltpu.make_async_copy(t8_hbm.at[pl.ds(off, CH)], buf.at[slot],
                                  seml.at[slot]).wait()

        def scat(i, slot):
            pltpu.async_copy(buf.at[slot], acc.at[idxv.at[i]],
                             sems.at[slot], add=True)

        def wait_scat(i, slot):
            pltpu.make_async_copy(buf.at[slot], acc.at[idxv.at[i]],
                                  sems.at[slot]).wait()

        fetch(0, 0)

        def step(i, carry):
            slot = lax.rem(i, 2)
            nslot = 1 - slot

            @pl.when(i + 1 < nchunk)
            def _():
                @pl.when(i >= 1)
                def _():
                    wait_scat(i - 1, nslot)

                fetch(i + 1, nslot)

            wait_fetch(i, slot)
            scat(i, slot)
            return carry

        lax.fori_loop(0, nchunk, step, 0)
        for k in (nchunk - 2, nchunk - 1):
            wait_scat(k, k % 2)
        plsc.subcore_barrier()

        @pl.when(s == 0)
        def _():
            pltpu.sync_copy(acc, out_hbm.at[c])

    fn = pl.kernel(
        body,
        out_type=jax.ShapeDtypeStruct((NC, n, 8), jnp.float32),
        mesh=mesh,
        compiler_params=pltpu.CompilerParams(use_tc_tiling_on_sc=False),
        scratch_types=[
            pltpu.VMEM((nchunk, CH), jnp.int32),
            pltpu.VMEM((2, CH, 8), jnp.float32),
            pltpu.VMEM_SHARED((n, 8), jnp.float32),
            pltpu.SemaphoreType.DMA((2,)),
            pltpu.SemaphoreType.DMA((2,)),
        ],
    )
    row4 = row.reshape(NC, NS, nchunk, CH)
    return fn(t8, row4, zer)


def _combine_body(p_ref, o_ref):
    p = p_ref[...]
    acc = jnp.sum(p, axis=0)
    cnt = jnp.maximum(acc[:, 3:4], 1.0)
    o_ref[...] = acc[:, 0:3] / cnt


def _combine(parts):
    _, n, _ = parts.shape
    return pl.pallas_call(
        _combine_body,
        out_shape=jax.ShapeDtypeStruct((n, 3), jnp.float32),
    )(parts)


NSLICE = 2


def kernel(h, coord_diff, edge_index, W1, b1, W2, b2, Wc1, bc1, Wc2):
    n, d = h.shape
    e = coord_diff.shape[0]
    row = edge_index[0]
    col = edge_index[1]
    w1a = W1[:d]
    w1b = W1[d:2 * d]
    w1c = W1[2 * d].reshape(1, d)
    ha, hb = _tables(h, w1a, w1b)
    zer = jnp.zeros((n, 8), jnp.float32)
    es = e // NSLICE
    parts = []
    for k in range(NSLICE):
        sl = slice(k * es, (k + 1) * es)
        pa, pb = _gather(ha, hb, row[sl], col[sl])
        t8 = _mlp(pa, pb, coord_diff[sl], w1c, b1.reshape(1, d), W2,
                  b2.reshape(1, d), Wc1, bc1.reshape(1, d), Wc2)
        parts.append(_scatter(t8, row[sl], zer))
    return _combine(jnp.concatenate(parts, axis=0))
